# Initial kernel scaffold; baseline (speedup 1.0000x reference)
import jax
import jax.numpy as jnp
from jax.experimental import pallas as pl


def kernel(x, edge_index, edge_attr, batch, lin0_w, lin0_b, nn_w1, nn_b1, nn_w2, nn_b2, conv_root, conv_bias, gru_w_ih, gru_w_hh, gru_b_ih, gru_b_hh, s2s_w_ih, s2s_w_hh, s2s_b_ih, s2s_b_hh, lin1_w, lin1_b, lin2_w, lin2_b):
    raise NotImplementedError("write your pallas kernel here")


# speedup vs baseline: 1.2682x; 1.2682x over previous
import functools

import jax
import jax.numpy as jnp
from jax import lax
from jax.experimental import pallas as pl
from jax.experimental.pallas import tpu as pltpu
from jax.experimental.pallas import tpu_sc as plsc

N = 8192
E = 16384
G = 512
D = 64
NFEAT = 29
EDIM = 6
DP = 128

HI = lax.Precision.HIGHEST

NC = 2
NS = 16
NW = NC * NS
EPW = E // NW
CHUNK = 128
NCHUNK = EPW // CHUNK


def _vec_mesh():
    return plsc.VectorSubcoreMesh(core_axis_name="c", subcore_axis_name="s")


def _sc_gather(table, idx):
    @functools.partial(
        pl.kernel,
        out_type=jax.ShapeDtypeStruct((E, DP), jnp.float32),
        mesh=_vec_mesh(),
        scratch_types=[
            pltpu.VMEM((CHUNK,), jnp.int32),
            pltpu.VMEM((CHUNK, DP), jnp.float32),
            pltpu.SemaphoreType.DMA,
        ],
    )
    def k(table_hbm, idx_hbm, out_hbm, idx_v, rows_v, sem):
        c = lax.axis_index("c")
        s = lax.axis_index("s")
        wid = s * NC + c
        base = wid * EPW
        for j in range(NCHUNK):
            off = base + j * CHUNK
            pltpu.sync_copy(idx_hbm.at[pl.ds(off, CHUNK)], idx_v)
            pltpu.async_copy(table_hbm.at[idx_v], rows_v, sem).wait()
            pltpu.sync_copy(rows_v, out_hbm.at[pl.ds(off, CHUNK)])

    return k(table, idx)


def _sc_scatter_add(vals, dst3, zeros):
    @functools.partial(
        pl.kernel,
        out_type=jax.ShapeDtypeStruct((NC * N, DP), jnp.float32),
        mesh=_vec_mesh(),
        scratch_types=[
            pltpu.VMEM((NCHUNK, CHUNK), jnp.int32),
            pltpu.VMEM((CHUNK, DP), jnp.float32),
            pltpu.VMEM_SHARED((N, DP), jnp.float32),
        ],
    )
    def k(vals_hbm, dst_hbm, zeros_hbm, out_hbm, idx_v, rows_v, acc_sh):
        c = lax.axis_index("c")
        s = lax.axis_index("s")
        wid = s * NC + c
        base = wid * EPW

        @pl.when(s == 0)
        def _():
            pltpu.sync_copy(zeros_hbm, acc_sh)

        plsc.subcore_barrier()
        pltpu.sync_copy(dst_hbm.at[wid], idx_v)
        for j in range(NCHUNK):
            pltpu.sync_copy(vals_hbm.at[pl.ds(base + j * CHUNK, CHUNK)], rows_v)
            pltpu.sync_copy(rows_v, acc_sh.at[idx_v.at[j]], add=True)
        plsc.subcore_barrier()
        rows_per_tile = N // NS
        pltpu.sync_copy(
            acc_sh.at[pl.ds(s * rows_per_tile, rows_per_tile)],
            out_hbm.at[pl.ds(c * N + s * rows_per_tile, rows_per_tile)],
        )

    return k(vals, dst3, zeros)


NBLK = 1024


def _lin0_body(x_ref, w_ref, b_ref, out_ref):
    o = jax.nn.relu(jnp.dot(x_ref[...], w_ref[...]) + b_ref[...])
    out_ref[:, 0:D] = o
    out_ref[:, D:DP] = jnp.zeros((NBLK, D), jnp.float32)


def _weh_body(ea_ref, w1_ref, b1_ref, weh_ref):
    weh_ref[...] = jax.nn.relu(
        jnp.dot(ea_ref[...], w1_ref[...]) + b1_ref[...]
    )


def _pre(x, lin0_w, lin0_b, edge_attr, nn_w1, nn_b1):
    out = pl.pallas_call(
        _lin0_body,
        grid=(N // NBLK,),
        in_specs=[
            pl.BlockSpec((NBLK, NFEAT), lambda i: (i, 0)),
            pl.BlockSpec((NFEAT, D), lambda i: (0, 0)),
            pl.BlockSpec((1, D), lambda i: (0, 0)),
        ],
        out_specs=pl.BlockSpec((NBLK, DP), lambda i: (i, 0)),
        out_shape=jax.ShapeDtypeStruct((N, DP), jnp.float32),
    )(x, lin0_w, lin0_b.reshape(1, D))
    weh = pl.pallas_call(
        _weh_body,
        grid=(E // (2 * NBLK),),
        in_specs=[
            pl.BlockSpec((2 * NBLK, EDIM), lambda i: (i, 0)),
            pl.BlockSpec((EDIM, 128), lambda i: (0, 0)),
            pl.BlockSpec((1, 128), lambda i: (0, 0)),
        ],
        out_specs=pl.BlockSpec((2 * NBLK, 128), lambda i: (i, 0)),
        out_shape=jax.ShapeDtypeStruct((E, 128), jnp.float32),
    )(edge_attr, nn_w1, nn_b1.reshape(1, 128))
    return out, weh


def _rcp_body(c0_ref, c1_ref, rcp_ref):
    rcp_ref[...] = 1.0 / jnp.maximum(c0_ref[...] + c1_ref[...], 1.0)


def _rcp(cntp):
    nb = N // NBLK
    return pl.pallas_call(
        _rcp_body,
        grid=(nb,),
        in_specs=[
            pl.BlockSpec((NBLK, DP), lambda i: (i, 0)),
            pl.BlockSpec((NBLK, DP), lambda i, _nb=nb: (i + _nb, 0)),
        ],
        out_specs=pl.BlockSpec((NBLK, DP), lambda i: (i, 0)),
        out_shape=jax.ShapeDtypeStruct((N, DP), jnp.float32),
    )(cntp, cntp)


EB = 512
KC = 512
IPC = KC // D


def _msg_body(weh_ref, w2_ref, b2_ref, xj_ref, msg_ref):
    xj = xj_ref[...]
    acc = jnp.zeros((EB, D), jnp.float32)
    for cidx in range(D * D // KC):
        wc = (
            jnp.dot(weh_ref[...], w2_ref[:, cidx * KC : (cidx + 1) * KC])
            + b2_ref[:, cidx * KC : (cidx + 1) * KC]
        )
        for t in range(IPC):
            i = cidx * IPC + t
            acc = acc + xj[:, i : i + 1] * wc[:, t * D : (t + 1) * D]
    msg_ref[:, 0:D] = acc
    msg_ref[:, D:DP] = jnp.zeros((EB, D), jnp.float32)


def _msg(weh, nn_w2, nn_b2, xj):
    grid = (E // EB,)
    return pl.pallas_call(
        _msg_body,
        grid=grid,
        in_specs=[
            pl.BlockSpec((EB, 128), lambda i: (i, 0)),
            pl.BlockSpec((128, D * D), lambda i: (0, 0)),
            pl.BlockSpec((1, D * D), lambda i: (0, 0)),
            pl.BlockSpec((EB, DP), lambda i: (i, 0)),
        ],
        out_specs=pl.BlockSpec((EB, DP), lambda i: (i, 0)),
        out_shape=jax.ShapeDtypeStruct((E, DP), jnp.float32),
    )(weh, nn_w2, nn_b2.reshape(1, D * D), xj)


def _gru_body(
    out_ref, a0_ref, a1_ref, rcp_ref, root_ref, cb_ref, wih_ref, whh_ref,
    bih_ref, bhh_ref, new_ref
):
    out = out_ref[:, 0:D]
    agg = (a0_ref[:, 0:D] + a1_ref[:, 0:D]) * rcp_ref[:, 0:D]
    m = jax.nn.relu(jnp.dot(out, root_ref[...]) + agg + cb_ref[...])
    gi = jnp.dot(m, wih_ref[...]) + bih_ref[...]
    gh = jnp.dot(out, whh_ref[...]) + bhh_ref[...]
    r = jax.nn.sigmoid(gi[:, 0:D] + gh[:, 0:D])
    z = jax.nn.sigmoid(gi[:, D : 2 * D] + gh[:, D : 2 * D])
    n = jnp.tanh(gi[:, 2 * D : 3 * D] + r * gh[:, 2 * D : 3 * D])
    new_ref[:, 0:D] = (1.0 - z) * n + z * out
    new_ref[:, D:DP] = jnp.zeros((NBLK, D), jnp.float32)


def _gru(out, aggp, rcp, conv_root, conv_bias, gru_w_ih, gru_w_hh, gru_b_ih, gru_b_hh):
    nb = N // NBLK
    blk = lambda i: (i, 0)
    cst = lambda i: (0, 0)
    return pl.pallas_call(
        _gru_body,
        grid=(nb,),
        in_specs=[
            pl.BlockSpec((NBLK, DP), blk),
            pl.BlockSpec((NBLK, DP), blk),
            pl.BlockSpec((NBLK, DP), lambda i, _nb=nb: (i + _nb, 0)),
            pl.BlockSpec((NBLK, DP), blk),
            pl.BlockSpec((D, D), cst),
            pl.BlockSpec((1, D), cst),
            pl.BlockSpec((D, 3 * D), cst),
            pl.BlockSpec((D, 3 * D), cst),
            pl.BlockSpec((1, 3 * D), cst),
            pl.BlockSpec((1, 3 * D), cst),
        ],
        out_specs=pl.BlockSpec((NBLK, DP), blk),
        out_shape=jax.ShapeDtypeStruct((N, DP), jnp.float32),
    )(
        out,
        aggp,
        aggp,
        rcp,
        conv_root,
        conv_bias.reshape(1, D),
        gru_w_ih,
        gru_w_hh,
        gru_b_ih.reshape(1, 3 * D),
        gru_b_hh.reshape(1, 3 * D),
    )


NCH = 8
CR = N // NCH


def _s2s_body(
    out_ref, bcol_ref, wih_ref, whh_ref, bih_ref, bhh_ref, l1w_ref,
    l1b_ref, l2w_ref, l2b_ref, y_ref, maskf, e2_s, ex_s
):
    iota_g = lax.broadcasted_iota(jnp.int32, (CR, G), 1)

    def _build(k, carry):
        rows = pl.ds(k * CR, CR)
        maskf[rows, :] = (bcol_ref[rows, :] == iota_g).astype(jnp.float32)
        return carry

    lax.fori_loop(0, NCH, _build, 0)

    q_star = jnp.zeros((G, 2 * D), jnp.float32)
    hh = jnp.zeros((G, D), jnp.float32)
    cc = jnp.zeros((G, D), jnp.float32)
    for _ in range(3):
        gates = (
            jnp.dot(q_star, wih_ref[...])
            + bih_ref[...]
            + jnp.dot(hh, whh_ref[...])
            + bhh_ref[...]
        )
        gi = jax.nn.sigmoid(gates[:, 0:D])
        gf = jax.nn.sigmoid(gates[:, D : 2 * D])
        gg = jnp.tanh(gates[:, 2 * D : 3 * D])
        go = jax.nn.sigmoid(gates[:, 3 * D : 4 * D])
        cc = gf * cc + gi * gg
        hh = go * jnp.tanh(cc)

        def _p1(k, mx):
            rows = pl.ds(k * CR, CR)
            mf = maskf[rows, :]
            qb = jnp.dot(mf, hh, precision=HI)
            e2 = jnp.sum(out_ref[rows, :] * qb, axis=1, keepdims=True)
            e2_s[rows, :] = e2
            masked = jnp.where(mf > 0.5, jnp.broadcast_to(e2, (CR, G)), -3.4e38)
            return jnp.maximum(mx, jnp.max(masked, axis=0, keepdims=True))

        mx = lax.fori_loop(0, NCH, _p1, jnp.full((1, G), -3.4e38, jnp.float32))

        def _p2(k, ssum):
            rows = pl.ds(k * CR, CR)
            mf = maskf[rows, :]
            mxb = jnp.sum(mf * mx, axis=1, keepdims=True)
            ex = jnp.exp(e2_s[rows, :] - mxb)
            ex_s[rows, :] = ex
            return ssum + jnp.sum(mf * ex, axis=0, keepdims=True)

        ssum = lax.fori_loop(0, NCH, _p2, jnp.zeros((1, G), jnp.float32))

        def _p3(k, r_read):
            rows = pl.ds(k * CR, CR)
            mf = maskf[rows, :]
            ssb = jnp.sum(mf * ssum, axis=1, keepdims=True)
            a = ex_s[rows, :] / (ssb + 1e-16)
            return r_read + lax.dot_general(
                mf, a * out_ref[rows, :], (((0,), (0,)), ((), ())), precision=HI
            )

        r_read = lax.fori_loop(0, NCH, _p3, jnp.zeros((G, D), jnp.float32))
        q_star = jnp.concatenate([hh, r_read], axis=1)

    y = jax.nn.relu(jnp.dot(q_star, l1w_ref[...]) + l1b_ref[...])
    y_ref[...] = jnp.dot(y, l2w_ref[...]) + l2b_ref[...]


def _s2s(out, batch, s2s_w_ih, s2s_w_hh, s2s_b_ih, s2s_b_hh, lin1_w, lin1_b,
         lin2_w, lin2_b):
    return pl.pallas_call(
        _s2s_body,
        out_shape=jax.ShapeDtypeStruct((G, 1), jnp.float32),
        scratch_shapes=[
            pltpu.VMEM((N, G), jnp.float32),
            pltpu.VMEM((N, 1), jnp.float32),
            pltpu.VMEM((N, 1), jnp.float32),
        ],
    )(
        out[:, 0:D],
        batch.reshape(N, 1),
        s2s_w_ih,
        s2s_w_hh,
        s2s_b_ih.reshape(1, 4 * D),
        s2s_b_hh.reshape(1, 4 * D),
        lin1_w,
        lin1_b.reshape(1, D),
        lin2_w,
        lin2_b.reshape(1, 1),
    )


def kernel(x, edge_index, edge_attr, batch, lin0_w, lin0_b, nn_w1, nn_b1,
           nn_w2, nn_b2, conv_root, conv_bias, gru_w_ih, gru_w_hh, gru_b_ih,
           gru_b_hh, s2s_w_ih, s2s_w_hh, s2s_b_ih, s2s_b_hh, lin1_w, lin1_b,
           lin2_w, lin2_b):
    src = edge_index[0]
    dst3 = edge_index[1].reshape(NW, NCHUNK, CHUNK)
    zeros_nd = jnp.zeros((N, DP), jnp.float32)

    out, weh = _pre(x, lin0_w, lin0_b, edge_attr, nn_w1, nn_b1)
    cntp = _sc_scatter_add(jnp.ones((E, DP), jnp.float32), dst3, zeros_nd)
    rcp = _rcp(cntp)

    for _ in range(6):
        xj = _sc_gather(out, src)
        msg = _msg(weh, nn_w2, nn_b2, xj)
        aggp = _sc_scatter_add(msg, dst3, zeros_nd)
        out = _gru(out, aggp, rcp, conv_root, conv_bias, gru_w_ih, gru_w_hh,
                   gru_b_ih, gru_b_hh)

    return _s2s(out, batch, s2s_w_ih, s2s_w_hh, s2s_b_ih, s2s_b_hh, lin1_w,
                lin1_b, lin2_w, lin2_b)


# speedup vs baseline: 2.3338x; 1.8403x over previous
import functools

import jax
import jax.numpy as jnp
from jax import lax
from jax.experimental import pallas as pl
from jax.experimental.pallas import tpu as pltpu
from jax.experimental.pallas import tpu_sc as plsc

N = 8192
E = 16384
G = 512
D = 64
NFEAT = 29
EDIM = 6
DP = 128

HI = lax.Precision.HIGHEST

NC = 2
NS = 16
NW = NC * NS
EPW = E // NW
CHUNK = 128
NCHUNK = EPW // CHUNK


def _vec_mesh():
    return plsc.VectorSubcoreMesh(core_axis_name="c", subcore_axis_name="s")


def _sc_gather(table, idx):
    @functools.partial(
        pl.kernel,
        out_type=jax.ShapeDtypeStruct((E, DP), jnp.float32),
        mesh=_vec_mesh(),
        scratch_types=[
            pltpu.VMEM((CHUNK,), jnp.int32),
            pltpu.VMEM((CHUNK, DP), jnp.float32),
            pltpu.SemaphoreType.DMA,
        ],
    )
    def k(table_hbm, idx_hbm, out_hbm, idx_v, rows_v, sem):
        c = lax.axis_index("c")
        s = lax.axis_index("s")
        wid = s * NC + c
        base = wid * EPW
        for j in range(NCHUNK):
            off = base + j * CHUNK
            pltpu.sync_copy(idx_hbm.at[pl.ds(off, CHUNK)], idx_v)
            pltpu.async_copy(table_hbm.at[idx_v], rows_v, sem).wait()
            pltpu.sync_copy(rows_v, out_hbm.at[pl.ds(off, CHUNK)])

    return k(table, idx)


def _sc_scatter_add(vals, dst3, zeros):
    @functools.partial(
        pl.kernel,
        out_type=jax.ShapeDtypeStruct((NC * N, DP), jnp.float32),
        mesh=_vec_mesh(),
        scratch_types=[
            pltpu.VMEM((NCHUNK, CHUNK), jnp.int32),
            pltpu.VMEM((CHUNK, DP), jnp.float32),
            pltpu.VMEM_SHARED((N, DP), jnp.float32),
        ],
    )
    def k(vals_hbm, dst_hbm, zeros_hbm, out_hbm, idx_v, rows_v, acc_sh):
        c = lax.axis_index("c")
        s = lax.axis_index("s")
        wid = s * NC + c
        base = wid * EPW

        @pl.when(s == 0)
        def _():
            pltpu.sync_copy(zeros_hbm, acc_sh)

        plsc.subcore_barrier()
        pltpu.sync_copy(dst_hbm.at[wid], idx_v)
        for j in range(NCHUNK):
            pltpu.sync_copy(vals_hbm.at[pl.ds(base + j * CHUNK, CHUNK)], rows_v)
            pltpu.sync_copy(rows_v, acc_sh.at[idx_v.at[j]], add=True)
        plsc.subcore_barrier()
        rows_per_tile = N // NS
        pltpu.sync_copy(
            acc_sh.at[pl.ds(s * rows_per_tile, rows_per_tile)],
            out_hbm.at[pl.ds(c * N + s * rows_per_tile, rows_per_tile)],
        )

    return k(vals, dst3, zeros)


NBLK = 1024


def _lin0_body(x_ref, w_ref, b_ref, out_ref):
    o = jax.nn.relu(jnp.dot(x_ref[...], w_ref[...]) + b_ref[...])
    out_ref[:, 0:D] = o
    out_ref[:, D:DP] = jnp.zeros((NBLK, D), jnp.float32)


def _weh_body(w1t_ref, eat_ref, b1_ref, weht_ref):
    weht_ref[...] = jax.nn.relu(
        jnp.dot(w1t_ref[...], eat_ref[...]) + b1_ref[...]
    )


def _pre(x, lin0_w, lin0_b, edge_attr, nn_w1, nn_b1):
    out = pl.pallas_call(
        _lin0_body,
        grid=(N // NBLK,),
        in_specs=[
            pl.BlockSpec((NBLK, NFEAT), lambda i: (i, 0)),
            pl.BlockSpec((NFEAT, D), lambda i: (0, 0)),
            pl.BlockSpec((1, D), lambda i: (0, 0)),
        ],
        out_specs=pl.BlockSpec((NBLK, DP), lambda i: (i, 0)),
        out_shape=jax.ShapeDtypeStruct((N, DP), jnp.float32),
    )(x, lin0_w, lin0_b.reshape(1, D))
    weht = pl.pallas_call(
        _weh_body,
        grid=(E // (2 * NBLK),),
        in_specs=[
            pl.BlockSpec((128, EDIM), lambda i: (0, 0)),
            pl.BlockSpec((EDIM, 2 * NBLK), lambda i: (0, i)),
            pl.BlockSpec((128, 1), lambda i: (0, 0)),
        ],
        out_specs=pl.BlockSpec((128, 2 * NBLK), lambda i: (0, i)),
        out_shape=jax.ShapeDtypeStruct((128, E), jnp.float32),
    )(nn_w1.T, edge_attr.T, nn_b1.reshape(128, 1))
    return out, weht


def _rcp_body(c0_ref, c1_ref, rcp_ref):
    rcp_ref[...] = 1.0 / jnp.maximum(c0_ref[...] + c1_ref[...], 1.0)


def _rcp(cntp):
    nb = N // NBLK
    return pl.pallas_call(
        _rcp_body,
        grid=(nb,),
        in_specs=[
            pl.BlockSpec((NBLK, DP), lambda i: (i, 0)),
            pl.BlockSpec((NBLK, DP), lambda i, _nb=nb: (i + _nb, 0)),
        ],
        out_specs=pl.BlockSpec((NBLK, DP), lambda i: (i, 0)),
        out_shape=jax.ShapeDtypeStruct((N, DP), jnp.float32),
    )(cntp, cntp)


EB = 512
KC = 512
IPC = KC // D


def _we_body(w2t_ref, weht_ref, wt_ref):
    wt_ref[...] = jnp.dot(w2t_ref[...], weht_ref[...])


def _we(weht, nn_w2):
    return pl.pallas_call(
        _we_body,
        grid=(E // EB,),
        in_specs=[
            pl.BlockSpec((D * D, 128), lambda i: (0, 0)),
            pl.BlockSpec((128, EB), lambda i: (0, i)),
        ],
        out_specs=pl.BlockSpec((D * D, EB), lambda i: (0, i)),
        out_shape=jax.ShapeDtypeStruct((D * D, E), jnp.float32),
    )(nn_w2.T, weht)


def _msg_body(wt_ref, xj_ref, b2t_ref, msg_ref):
    xjt = jnp.transpose(xj_ref[...])
    acc = jnp.dot(b2t_ref[...], xjt[0:D, :], precision=HI)
    for i in range(D):
        xr = jnp.broadcast_to(xjt[i : i + 1, :], (D, EB))
        acc = acc + xr * wt_ref[i * D : (i + 1) * D, :]
    msg_ref[:, 0:D] = jnp.transpose(acc)
    msg_ref[:, D:DP] = jnp.zeros((EB, D), jnp.float32)


def _msg(wt, xj, b2t):
    return pl.pallas_call(
        _msg_body,
        grid=(E // EB,),
        in_specs=[
            pl.BlockSpec((D * D, EB), lambda i: (0, i)),
            pl.BlockSpec((EB, DP), lambda i: (i, 0)),
            pl.BlockSpec((D, D), lambda i: (0, 0)),
        ],
        out_specs=pl.BlockSpec((EB, DP), lambda i: (i, 0)),
        out_shape=jax.ShapeDtypeStruct((E, DP), jnp.float32),
    )(wt, xj, b2t)


def _gru_body(
    out_ref, a0_ref, a1_ref, rcp_ref, root_ref, cb_ref, wih_ref, whh_ref,
    bih_ref, bhh_ref, new_ref
):
    out = out_ref[:, 0:D]
    agg = (a0_ref[:, 0:D] + a1_ref[:, 0:D]) * rcp_ref[:, 0:D]
    m = jax.nn.relu(jnp.dot(out, root_ref[...]) + agg + cb_ref[...])
    gi = jnp.dot(m, wih_ref[...]) + bih_ref[...]
    gh = jnp.dot(out, whh_ref[...]) + bhh_ref[...]
    r = jax.nn.sigmoid(gi[:, 0:D] + gh[:, 0:D])
    z = jax.nn.sigmoid(gi[:, D : 2 * D] + gh[:, D : 2 * D])
    n = jnp.tanh(gi[:, 2 * D : 3 * D] + r * gh[:, 2 * D : 3 * D])
    new_ref[:, 0:D] = (1.0 - z) * n + z * out
    new_ref[:, D:DP] = jnp.zeros((NBLK, D), jnp.float32)


def _gru(out, aggp, rcp, conv_root, conv_bias, gru_w_ih, gru_w_hh, gru_b_ih, gru_b_hh):
    nb = N // NBLK
    blk = lambda i: (i, 0)
    cst = lambda i: (0, 0)
    return pl.pallas_call(
        _gru_body,
        grid=(nb,),
        in_specs=[
            pl.BlockSpec((NBLK, DP), blk),
            pl.BlockSpec((NBLK, DP), blk),
            pl.BlockSpec((NBLK, DP), lambda i, _nb=nb: (i + _nb, 0)),
            pl.BlockSpec((NBLK, DP), blk),
            pl.BlockSpec((D, D), cst),
            pl.BlockSpec((1, D), cst),
            pl.BlockSpec((D, 3 * D), cst),
            pl.BlockSpec((D, 3 * D), cst),
            pl.BlockSpec((1, 3 * D), cst),
            pl.BlockSpec((1, 3 * D), cst),
        ],
        out_specs=pl.BlockSpec((NBLK, DP), blk),
        out_shape=jax.ShapeDtypeStruct((N, DP), jnp.float32),
    )(
        out,
        aggp,
        aggp,
        rcp,
        conv_root,
        conv_bias.reshape(1, D),
        gru_w_ih,
        gru_w_hh,
        gru_b_ih.reshape(1, 3 * D),
        gru_b_hh.reshape(1, 3 * D),
    )


NCH = 8
CR = N // NCH


def _s2s_body(
    out_ref, bcol_ref, wih_ref, whh_ref, bih_ref, bhh_ref, l1w_ref,
    l1b_ref, l2w_ref, l2b_ref, y_ref, maskf, e2_s, ex_s
):
    iota_g = lax.broadcasted_iota(jnp.int32, (CR, G), 1)

    def _build(k, carry):
        rows = pl.ds(k * CR, CR)
        maskf[rows, :] = (bcol_ref[rows, :] == iota_g).astype(jnp.float32)
        return carry

    lax.fori_loop(0, NCH, _build, 0)

    q_star = jnp.zeros((G, 2 * D), jnp.float32)
    hh = jnp.zeros((G, D), jnp.float32)
    cc = jnp.zeros((G, D), jnp.float32)
    for _ in range(3):
        gates = (
            jnp.dot(q_star, wih_ref[...])
            + bih_ref[...]
            + jnp.dot(hh, whh_ref[...])
            + bhh_ref[...]
        )
        gi = jax.nn.sigmoid(gates[:, 0:D])
        gf = jax.nn.sigmoid(gates[:, D : 2 * D])
        gg = jnp.tanh(gates[:, 2 * D : 3 * D])
        go = jax.nn.sigmoid(gates[:, 3 * D : 4 * D])
        cc = gf * cc + gi * gg
        hh = go * jnp.tanh(cc)

        def _p1(k, mx):
            rows = pl.ds(k * CR, CR)
            mf = maskf[rows, :]
            qb = jnp.dot(mf, hh, precision=HI)
            e2 = jnp.sum(out_ref[rows, :] * qb, axis=1, keepdims=True)
            e2_s[rows, :] = e2
            masked = jnp.where(mf > 0.5, jnp.broadcast_to(e2, (CR, G)), -3.4e38)
            return jnp.maximum(mx, jnp.max(masked, axis=0, keepdims=True))

        mx = lax.fori_loop(0, NCH, _p1, jnp.full((1, G), -3.4e38, jnp.float32))

        def _p2(k, ssum):
            rows = pl.ds(k * CR, CR)
            mf = maskf[rows, :]
            mxb = jnp.sum(mf * mx, axis=1, keepdims=True)
            ex = jnp.exp(e2_s[rows, :] - mxb)
            ex_s[rows, :] = ex
            return ssum + jnp.sum(mf * ex, axis=0, keepdims=True)

        ssum = lax.fori_loop(0, NCH, _p2, jnp.zeros((1, G), jnp.float32))

        def _p3(k, r_read):
            rows = pl.ds(k * CR, CR)
            mf = maskf[rows, :]
            ssb = jnp.sum(mf * ssum, axis=1, keepdims=True)
            a = ex_s[rows, :] / (ssb + 1e-16)
            return r_read + lax.dot_general(
                mf, a * out_ref[rows, :], (((0,), (0,)), ((), ())), precision=HI
            )

        r_read = lax.fori_loop(0, NCH, _p3, jnp.zeros((G, D), jnp.float32))
        q_star = jnp.concatenate([hh, r_read], axis=1)

    y = jax.nn.relu(jnp.dot(q_star, l1w_ref[...]) + l1b_ref[...])
    y_ref[...] = jnp.dot(y, l2w_ref[...]) + l2b_ref[...]


def _s2s(out, batch, s2s_w_ih, s2s_w_hh, s2s_b_ih, s2s_b_hh, lin1_w, lin1_b,
         lin2_w, lin2_b):
    return pl.pallas_call(
        _s2s_body,
        out_shape=jax.ShapeDtypeStruct((G, 1), jnp.float32),
        scratch_shapes=[
            pltpu.VMEM((N, G), jnp.float32),
            pltpu.VMEM((N, 1), jnp.float32),
            pltpu.VMEM((N, 1), jnp.float32),
        ],
    )(
        out[:, 0:D],
        batch.reshape(N, 1),
        s2s_w_ih,
        s2s_w_hh,
        s2s_b_ih.reshape(1, 4 * D),
        s2s_b_hh.reshape(1, 4 * D),
        lin1_w,
        lin1_b.reshape(1, D),
        lin2_w,
        lin2_b.reshape(1, 1),
    )


def kernel(x, edge_index, edge_attr, batch, lin0_w, lin0_b, nn_w1, nn_b1,
           nn_w2, nn_b2, conv_root, conv_bias, gru_w_ih, gru_w_hh, gru_b_ih,
           gru_b_hh, s2s_w_ih, s2s_w_hh, s2s_b_ih, s2s_b_hh, lin1_w, lin1_b,
           lin2_w, lin2_b):
    src = edge_index[0]
    dst3 = edge_index[1].reshape(NW, NCHUNK, CHUNK)
    zeros_nd = jnp.zeros((N, DP), jnp.float32)

    out, weht = _pre(x, lin0_w, lin0_b, edge_attr, nn_w1, nn_b1)
    wt = _we(weht, nn_w2)
    b2t = nn_b2.reshape(D, D).T
    cntp = _sc_scatter_add(jnp.ones((E, DP), jnp.float32), dst3, zeros_nd)
    rcp = _rcp(cntp)

    for _ in range(6):
        xj = _sc_gather(out, src)
        msg = _msg(wt, xj, b2t)
        aggp = _sc_scatter_add(msg, dst3, zeros_nd)
        out = _gru(out, aggp, rcp, conv_root, conv_bias, gru_w_ih, gru_w_hh,
                   gru_b_ih, gru_b_hh)

    return _s2s(out, batch, s2s_w_ih, s2s_w_hh, s2s_b_ih, s2s_b_hh, lin1_w,
                lin1_b, lin2_w, lin2_b)


# speedup vs baseline: 2.3344x; 1.0003x over previous
import functools

import jax
import jax.numpy as jnp
from jax import lax
from jax.experimental import pallas as pl
from jax.experimental.pallas import tpu as pltpu
from jax.experimental.pallas import tpu_sc as plsc

N = 8192
E = 16384
G = 512
D = 64
NFEAT = 29
EDIM = 6
DP = 128

HI = lax.Precision.HIGHEST
PAR = pltpu.CompilerParams(dimension_semantics=("parallel",))

NC = 2
NS = 16
NW = NC * NS
EPW = E // NW
CHUNK = 128
NCHUNK = EPW // CHUNK


def _vec_mesh():
    return plsc.VectorSubcoreMesh(core_axis_name="c", subcore_axis_name="s")


def _sc_gather(table, idx):
    @functools.partial(
        pl.kernel,
        out_type=jax.ShapeDtypeStruct((E, DP), jnp.float32),
        mesh=_vec_mesh(),
        scratch_types=[
            pltpu.VMEM((CHUNK,), jnp.int32),
            pltpu.VMEM((CHUNK, DP), jnp.float32),
            pltpu.SemaphoreType.DMA,
        ],
    )
    def k(table_hbm, idx_hbm, out_hbm, idx_v, rows_v, sem):
        c = lax.axis_index("c")
        s = lax.axis_index("s")
        wid = s * NC + c
        base = wid * EPW
        for j in range(NCHUNK):
            off = base + j * CHUNK
            pltpu.sync_copy(idx_hbm.at[pl.ds(off, CHUNK)], idx_v)
            pltpu.async_copy(table_hbm.at[idx_v], rows_v, sem).wait()
            pltpu.sync_copy(rows_v, out_hbm.at[pl.ds(off, CHUNK)])

    return k(table, idx)


def _sc_scatter_add(vals, dst3, zeros):
    @functools.partial(
        pl.kernel,
        out_type=jax.ShapeDtypeStruct((NC * N, DP), jnp.float32),
        mesh=_vec_mesh(),
        scratch_types=[
            pltpu.VMEM((NCHUNK, CHUNK), jnp.int32),
            pltpu.VMEM((CHUNK, DP), jnp.float32),
            pltpu.VMEM_SHARED((N, DP), jnp.float32),
        ],
    )
    def k(vals_hbm, dst_hbm, zeros_hbm, out_hbm, idx_v, rows_v, acc_sh):
        c = lax.axis_index("c")
        s = lax.axis_index("s")
        wid = s * NC + c
        base = wid * EPW

        @pl.when(s == 0)
        def _():
            pltpu.sync_copy(zeros_hbm, acc_sh)

        plsc.subcore_barrier()
        pltpu.sync_copy(dst_hbm.at[wid], idx_v)
        for j in range(NCHUNK):
            pltpu.sync_copy(vals_hbm.at[pl.ds(base + j * CHUNK, CHUNK)], rows_v)
            pltpu.sync_copy(rows_v, acc_sh.at[idx_v.at[j]], add=True)
        plsc.subcore_barrier()
        rows_per_tile = N // NS
        pltpu.sync_copy(
            acc_sh.at[pl.ds(s * rows_per_tile, rows_per_tile)],
            out_hbm.at[pl.ds(c * N + s * rows_per_tile, rows_per_tile)],
        )

    return k(vals, dst3, zeros)


NBLK = 1024


def _lin0_body(x_ref, w_ref, b_ref, out_ref):
    o = jax.nn.relu(jnp.dot(x_ref[...], w_ref[...]) + b_ref[...])
    out_ref[:, 0:D] = o
    out_ref[:, D:DP] = jnp.zeros((NBLK, D), jnp.float32)


def _weh_body(w1t_ref, eat_ref, b1_ref, weht_ref):
    weht_ref[...] = jax.nn.relu(
        jnp.dot(w1t_ref[...], eat_ref[...]) + b1_ref[...]
    )


def _pre(x, lin0_w, lin0_b, edge_attr, nn_w1, nn_b1):
    out = pl.pallas_call(
        _lin0_body,
        compiler_params=PAR,
        grid=(N // NBLK,),
        in_specs=[
            pl.BlockSpec((NBLK, NFEAT), lambda i: (i, 0)),
            pl.BlockSpec((NFEAT, D), lambda i: (0, 0)),
            pl.BlockSpec((1, D), lambda i: (0, 0)),
        ],
        out_specs=pl.BlockSpec((NBLK, DP), lambda i: (i, 0)),
        out_shape=jax.ShapeDtypeStruct((N, DP), jnp.float32),
    )(x, lin0_w, lin0_b.reshape(1, D))
    weht = pl.pallas_call(
        _weh_body,
        compiler_params=PAR,
        grid=(E // (2 * NBLK),),
        in_specs=[
            pl.BlockSpec((128, EDIM), lambda i: (0, 0)),
            pl.BlockSpec((EDIM, 2 * NBLK), lambda i: (0, i)),
            pl.BlockSpec((128, 1), lambda i: (0, 0)),
        ],
        out_specs=pl.BlockSpec((128, 2 * NBLK), lambda i: (0, i)),
        out_shape=jax.ShapeDtypeStruct((128, E), jnp.float32),
    )(nn_w1.T, edge_attr.T, nn_b1.reshape(128, 1))
    return out, weht


def _rcp_body(c0_ref, c1_ref, rcp_ref):
    rcp_ref[...] = 1.0 / jnp.maximum(c0_ref[...] + c1_ref[...], 1.0)


def _rcp(cntp):
    nb = N // NBLK
    return pl.pallas_call(
        _rcp_body,
        compiler_params=PAR,
        grid=(nb,),
        in_specs=[
            pl.BlockSpec((NBLK, DP), lambda i: (i, 0)),
            pl.BlockSpec((NBLK, DP), lambda i, _nb=nb: (i + _nb, 0)),
        ],
        out_specs=pl.BlockSpec((NBLK, DP), lambda i: (i, 0)),
        out_shape=jax.ShapeDtypeStruct((N, DP), jnp.float32),
    )(cntp, cntp)


EB = 512
KC = 512
IPC = KC // D


def _we_body(w2t_ref, weht_ref, wt_ref):
    wt_ref[...] = jnp.dot(w2t_ref[...], weht_ref[...])


def _we(weht, nn_w2):
    return pl.pallas_call(
        _we_body,
        compiler_params=PAR,
        grid=(E // EB,),
        in_specs=[
            pl.BlockSpec((D * D, 128), lambda i: (0, 0)),
            pl.BlockSpec((128, EB), lambda i: (0, i)),
        ],
        out_specs=pl.BlockSpec((D * D, EB), lambda i: (0, i)),
        out_shape=jax.ShapeDtypeStruct((D * D, E), jnp.float32),
    )(nn_w2.T, weht)


def _msg_body(wt_ref, xj_ref, b2t_ref, msg_ref):
    xjt = jnp.transpose(xj_ref[...])
    acc = jnp.dot(b2t_ref[...], xjt[0:D, :], precision=HI)
    for i in range(D):
        xr = jnp.broadcast_to(xjt[i : i + 1, :], (D, EB))
        acc = acc + xr * wt_ref[i * D : (i + 1) * D, :]
    msg_ref[:, 0:D] = jnp.transpose(acc)
    msg_ref[:, D:DP] = jnp.zeros((EB, D), jnp.float32)


def _msg(wt, xj, b2t):
    return pl.pallas_call(
        _msg_body,
        compiler_params=PAR,
        grid=(E // EB,),
        in_specs=[
            pl.BlockSpec((D * D, EB), lambda i: (0, i)),
            pl.BlockSpec((EB, DP), lambda i: (i, 0)),
            pl.BlockSpec((D, D), lambda i: (0, 0)),
        ],
        out_specs=pl.BlockSpec((EB, DP), lambda i: (i, 0)),
        out_shape=jax.ShapeDtypeStruct((E, DP), jnp.float32),
    )(wt, xj, b2t)


def _gru_body(
    out_ref, a0_ref, a1_ref, rcp_ref, root_ref, cb_ref, wih_ref, whh_ref,
    bih_ref, bhh_ref, new_ref
):
    out = out_ref[:, 0:D]
    agg = (a0_ref[:, 0:D] + a1_ref[:, 0:D]) * rcp_ref[:, 0:D]
    m = jax.nn.relu(jnp.dot(out, root_ref[...]) + agg + cb_ref[...])
    gi = jnp.dot(m, wih_ref[...]) + bih_ref[...]
    gh = jnp.dot(out, whh_ref[...]) + bhh_ref[...]
    r = jax.nn.sigmoid(gi[:, 0:D] + gh[:, 0:D])
    z = jax.nn.sigmoid(gi[:, D : 2 * D] + gh[:, D : 2 * D])
    n = jnp.tanh(gi[:, 2 * D : 3 * D] + r * gh[:, 2 * D : 3 * D])
    new_ref[:, 0:D] = (1.0 - z) * n + z * out
    new_ref[:, D:DP] = jnp.zeros((NBLK, D), jnp.float32)


def _gru(out, aggp, rcp, conv_root, conv_bias, gru_w_ih, gru_w_hh, gru_b_ih, gru_b_hh):
    nb = N // NBLK
    blk = lambda i: (i, 0)
    cst = lambda i: (0, 0)
    return pl.pallas_call(
        _gru_body,
        compiler_params=PAR,
        grid=(nb,),
        in_specs=[
            pl.BlockSpec((NBLK, DP), blk),
            pl.BlockSpec((NBLK, DP), blk),
            pl.BlockSpec((NBLK, DP), lambda i, _nb=nb: (i + _nb, 0)),
            pl.BlockSpec((NBLK, DP), blk),
            pl.BlockSpec((D, D), cst),
            pl.BlockSpec((1, D), cst),
            pl.BlockSpec((D, 3 * D), cst),
            pl.BlockSpec((D, 3 * D), cst),
            pl.BlockSpec((1, 3 * D), cst),
            pl.BlockSpec((1, 3 * D), cst),
        ],
        out_specs=pl.BlockSpec((NBLK, DP), blk),
        out_shape=jax.ShapeDtypeStruct((N, DP), jnp.float32),
    )(
        out,
        aggp,
        aggp,
        rcp,
        conv_root,
        conv_bias.reshape(1, D),
        gru_w_ih,
        gru_w_hh,
        gru_b_ih.reshape(1, 3 * D),
        gru_b_hh.reshape(1, 3 * D),
    )


NCH = 8
CR = N // NCH


def _s2s_body(
    out_ref, bcol_ref, wih_ref, whh_ref, bih_ref, bhh_ref, l1w_ref,
    l1b_ref, l2w_ref, l2b_ref, y_ref, maskf, e2_s, ex_s
):
    iota_g = lax.broadcasted_iota(jnp.int32, (CR, G), 1)

    def _build(k, carry):
        rows = pl.ds(k * CR, CR)
        maskf[rows, :] = (bcol_ref[rows, :] == iota_g).astype(jnp.float32)
        return carry

    lax.fori_loop(0, NCH, _build, 0)

    q_star = jnp.zeros((G, 2 * D), jnp.float32)
    hh = jnp.zeros((G, D), jnp.float32)
    cc = jnp.zeros((G, D), jnp.float32)
    for _ in range(3):
        gates = (
            jnp.dot(q_star, wih_ref[...])
            + bih_ref[...]
            + jnp.dot(hh, whh_ref[...])
            + bhh_ref[...]
        )
        gi = jax.nn.sigmoid(gates[:, 0:D])
        gf = jax.nn.sigmoid(gates[:, D : 2 * D])
        gg = jnp.tanh(gates[:, 2 * D : 3 * D])
        go = jax.nn.sigmoid(gates[:, 3 * D : 4 * D])
        cc = gf * cc + gi * gg
        hh = go * jnp.tanh(cc)

        def _p1(k, mx):
            rows = pl.ds(k * CR, CR)
            mf = maskf[rows, :]
            qb = jnp.dot(mf, hh, precision=HI)
            e2 = jnp.sum(out_ref[rows, :] * qb, axis=1, keepdims=True)
            e2_s[rows, :] = e2
            masked = jnp.where(mf > 0.5, jnp.broadcast_to(e2, (CR, G)), -3.4e38)
            return jnp.maximum(mx, jnp.max(masked, axis=0, keepdims=True))

        mx = lax.fori_loop(0, NCH, _p1, jnp.full((1, G), -3.4e38, jnp.float32))

        def _p2(k, ssum):
            rows = pl.ds(k * CR, CR)
            mf = maskf[rows, :]
            mxb = jnp.sum(mf * mx, axis=1, keepdims=True)
            ex = jnp.exp(e2_s[rows, :] - mxb)
            ex_s[rows, :] = ex
            return ssum + jnp.sum(mf * ex, axis=0, keepdims=True)

        ssum = lax.fori_loop(0, NCH, _p2, jnp.zeros((1, G), jnp.float32))

        def _p3(k, r_read):
            rows = pl.ds(k * CR, CR)
            mf = maskf[rows, :]
            ssb = jnp.sum(mf * ssum, axis=1, keepdims=True)
            a = ex_s[rows, :] / (ssb + 1e-16)
            return r_read + lax.dot_general(
                mf, a * out_ref[rows, :], (((0,), (0,)), ((), ())), precision=HI
            )

        r_read = lax.fori_loop(0, NCH, _p3, jnp.zeros((G, D), jnp.float32))
        q_star = jnp.concatenate([hh, r_read], axis=1)

    y = jax.nn.relu(jnp.dot(q_star, l1w_ref[...]) + l1b_ref[...])
    y_ref[...] = jnp.dot(y, l2w_ref[...]) + l2b_ref[...]


def _s2s(out, batch, s2s_w_ih, s2s_w_hh, s2s_b_ih, s2s_b_hh, lin1_w, lin1_b,
         lin2_w, lin2_b):
    return pl.pallas_call(
        _s2s_body,
        out_shape=jax.ShapeDtypeStruct((G, 1), jnp.float32),
        scratch_shapes=[
            pltpu.VMEM((N, G), jnp.float32),
            pltpu.VMEM((N, 1), jnp.float32),
            pltpu.VMEM((N, 1), jnp.float32),
        ],
    )(
        out[:, 0:D],
        batch.reshape(N, 1),
        s2s_w_ih,
        s2s_w_hh,
        s2s_b_ih.reshape(1, 4 * D),
        s2s_b_hh.reshape(1, 4 * D),
        lin1_w,
        lin1_b.reshape(1, D),
        lin2_w,
        lin2_b.reshape(1, 1),
    )


def kernel(x, edge_index, edge_attr, batch, lin0_w, lin0_b, nn_w1, nn_b1,
           nn_w2, nn_b2, conv_root, conv_bias, gru_w_ih, gru_w_hh, gru_b_ih,
           gru_b_hh, s2s_w_ih, s2s_w_hh, s2s_b_ih, s2s_b_hh, lin1_w, lin1_b,
           lin2_w, lin2_b):
    src = edge_index[0]
    dst3 = edge_index[1].reshape(NW, NCHUNK, CHUNK)
    zeros_nd = jnp.zeros((N, DP), jnp.float32)

    out, weht = _pre(x, lin0_w, lin0_b, edge_attr, nn_w1, nn_b1)
    wt = _we(weht, nn_w2)
    b2t = nn_b2.reshape(D, D).T
    cntp = _sc_scatter_add(jnp.ones((E, DP), jnp.float32), dst3, zeros_nd)
    rcp = _rcp(cntp)

    for _ in range(6):
        xj = _sc_gather(out, src)
        msg = _msg(wt, xj, b2t)
        aggp = _sc_scatter_add(msg, dst3, zeros_nd)
        out = _gru(out, aggp, rcp, conv_root, conv_bias, gru_w_ih, gru_w_hh,
                   gru_b_ih, gru_b_hh)

    return _s2s(out, batch, s2s_w_ih, s2s_w_hh, s2s_b_ih, s2s_b_hh, lin1_w,
                lin1_b, lin2_w, lin2_b)


# speedup vs baseline: 2.3912x; 1.0243x over previous
import functools

import jax
import jax.numpy as jnp
from jax import lax
from jax.experimental import pallas as pl
from jax.experimental.pallas import tpu as pltpu
from jax.experimental.pallas import tpu_sc as plsc

N = 8192
E = 16384
G = 512
D = 64
NFEAT = 29
EDIM = 6
DP = 128

HI = lax.Precision.HIGHEST
PAR = pltpu.CompilerParams(dimension_semantics=("parallel",))

NC = 2
NS = 16
NW = NC * NS
EPW = E // NW
CHUNK = 128
NCHUNK = EPW // CHUNK


def _vec_mesh():
    return plsc.VectorSubcoreMesh(core_axis_name="c", subcore_axis_name="s")


def _sc_gather(table, idx):
    @functools.partial(
        pl.kernel,
        out_type=jax.ShapeDtypeStruct((E, DP), jnp.float32),
        mesh=_vec_mesh(),
        scratch_types=[
            pltpu.VMEM((EPW,), jnp.int32),
            pltpu.VMEM((NCHUNK, CHUNK, DP), jnp.float32),
            pltpu.SemaphoreType.DMA,
            pltpu.SemaphoreType.DMA,
        ],
    )
    def k(table_hbm, idx_hbm, out_hbm, idx_v, rows_v, gsem, ssem):
        c = lax.axis_index("c")
        s = lax.axis_index("s")
        wid = s * NC + c
        base = wid * EPW
        pltpu.sync_copy(idx_hbm.at[pl.ds(base, EPW)], idx_v)
        gathers = [
            pltpu.async_copy(
                table_hbm.at[idx_v.at[pl.ds(j * CHUNK, CHUNK)]], rows_v.at[j], gsem
            )
            for j in range(NCHUNK)
        ]
        stores = []
        for j in range(NCHUNK):
            gathers[j].wait()
            stores.append(
                pltpu.async_copy(
                    rows_v.at[j], out_hbm.at[pl.ds(base + j * CHUNK, CHUNK)], ssem
                )
            )
        for st in stores:
            st.wait()

    return k(table, idx)


def _sc_scatter_add(vals, dst3, zeros):
    @functools.partial(
        pl.kernel,
        out_type=jax.ShapeDtypeStruct((NC * N, DP), jnp.float32),
        mesh=_vec_mesh(),
        scratch_types=[
            pltpu.VMEM((NCHUNK, CHUNK), jnp.int32),
            pltpu.VMEM((CHUNK, DP), jnp.float32),
            pltpu.VMEM_SHARED((N, DP), jnp.float32),
        ],
    )
    def k(vals_hbm, dst_hbm, zeros_hbm, out_hbm, idx_v, rows_v, acc_sh):
        c = lax.axis_index("c")
        s = lax.axis_index("s")
        wid = s * NC + c
        base = wid * EPW

        @pl.when(s == 0)
        def _():
            pltpu.sync_copy(zeros_hbm, acc_sh)

        plsc.subcore_barrier()
        pltpu.sync_copy(dst_hbm.at[wid], idx_v)
        for j in range(NCHUNK):
            pltpu.sync_copy(vals_hbm.at[pl.ds(base + j * CHUNK, CHUNK)], rows_v)
            pltpu.sync_copy(rows_v, acc_sh.at[idx_v.at[j]], add=True)
        plsc.subcore_barrier()
        rows_per_tile = N // NS
        pltpu.sync_copy(
            acc_sh.at[pl.ds(s * rows_per_tile, rows_per_tile)],
            out_hbm.at[pl.ds(c * N + s * rows_per_tile, rows_per_tile)],
        )

    return k(vals, dst3, zeros)


NBLK = 1024


def _lin0_body(x_ref, w_ref, b_ref, out_ref):
    o = jax.nn.relu(jnp.dot(x_ref[...], w_ref[...]) + b_ref[...])
    out_ref[:, 0:D] = o
    out_ref[:, D:DP] = jnp.zeros((NBLK, D), jnp.float32)


def _weh_body(w1t_ref, eat_ref, b1_ref, weht_ref):
    weht_ref[...] = jax.nn.relu(
        jnp.dot(w1t_ref[...], eat_ref[...]) + b1_ref[...]
    )


def _pre(x, lin0_w, lin0_b, edge_attr, nn_w1, nn_b1):
    out = pl.pallas_call(
        _lin0_body,
        compiler_params=PAR,
        grid=(N // NBLK,),
        in_specs=[
            pl.BlockSpec((NBLK, NFEAT), lambda i: (i, 0)),
            pl.BlockSpec((NFEAT, D), lambda i: (0, 0)),
            pl.BlockSpec((1, D), lambda i: (0, 0)),
        ],
        out_specs=pl.BlockSpec((NBLK, DP), lambda i: (i, 0)),
        out_shape=jax.ShapeDtypeStruct((N, DP), jnp.float32),
    )(x, lin0_w, lin0_b.reshape(1, D))
    weht = pl.pallas_call(
        _weh_body,
        compiler_params=PAR,
        grid=(E // (2 * NBLK),),
        in_specs=[
            pl.BlockSpec((128, EDIM), lambda i: (0, 0)),
            pl.BlockSpec((EDIM, 2 * NBLK), lambda i: (0, i)),
            pl.BlockSpec((128, 1), lambda i: (0, 0)),
        ],
        out_specs=pl.BlockSpec((128, 2 * NBLK), lambda i: (0, i)),
        out_shape=jax.ShapeDtypeStruct((128, E), jnp.float32),
    )(nn_w1.T, edge_attr.T, nn_b1.reshape(128, 1))
    return out, weht


def _rcp_body(c0_ref, c1_ref, rcp_ref):
    rcp_ref[...] = 1.0 / jnp.maximum(c0_ref[...] + c1_ref[...], 1.0)


def _rcp(cntp):
    nb = N // NBLK
    return pl.pallas_call(
        _rcp_body,
        compiler_params=PAR,
        grid=(nb,),
        in_specs=[
            pl.BlockSpec((NBLK, DP), lambda i: (i, 0)),
            pl.BlockSpec((NBLK, DP), lambda i, _nb=nb: (i + _nb, 0)),
        ],
        out_specs=pl.BlockSpec((NBLK, DP), lambda i: (i, 0)),
        out_shape=jax.ShapeDtypeStruct((N, DP), jnp.float32),
    )(cntp, cntp)


EB = 512
KC = 512
IPC = KC // D


def _we_body(w2t_ref, weht_ref, wt_ref):
    wt_ref[...] = jnp.dot(w2t_ref[...], weht_ref[...])


def _we(weht, nn_w2):
    return pl.pallas_call(
        _we_body,
        compiler_params=PAR,
        grid=(E // EB,),
        in_specs=[
            pl.BlockSpec((D * D, 128), lambda i: (0, 0)),
            pl.BlockSpec((128, EB), lambda i: (0, i)),
        ],
        out_specs=pl.BlockSpec((D * D, EB), lambda i: (0, i)),
        out_shape=jax.ShapeDtypeStruct((D * D, E), jnp.float32),
    )(nn_w2.T, weht)


def _msg_body(wt_ref, xj_ref, b2t_ref, msg_ref):
    xjt = jnp.transpose(xj_ref[...])
    acc = jnp.dot(b2t_ref[...], xjt[0:D, :], precision=HI)
    for i in range(D):
        xr = jnp.broadcast_to(xjt[i : i + 1, :], (D, EB))
        acc = acc + xr * wt_ref[i * D : (i + 1) * D, :]
    msg_ref[:, 0:D] = jnp.transpose(acc)
    msg_ref[:, D:DP] = jnp.zeros((EB, D), jnp.float32)


def _msg(wt, xj, b2t):
    return pl.pallas_call(
        _msg_body,
        compiler_params=PAR,
        grid=(E // EB,),
        in_specs=[
            pl.BlockSpec((D * D, EB), lambda i: (0, i)),
            pl.BlockSpec((EB, DP), lambda i: (i, 0)),
            pl.BlockSpec((D, D), lambda i: (0, 0)),
        ],
        out_specs=pl.BlockSpec((EB, DP), lambda i: (i, 0)),
        out_shape=jax.ShapeDtypeStruct((E, DP), jnp.float32),
    )(wt, xj, b2t)


def _gru_body(
    out_ref, a0_ref, a1_ref, rcp_ref, root_ref, cb_ref, wih_ref, whh_ref,
    bih_ref, bhh_ref, new_ref
):
    out = out_ref[:, 0:D]
    agg = (a0_ref[:, 0:D] + a1_ref[:, 0:D]) * rcp_ref[:, 0:D]
    m = jax.nn.relu(jnp.dot(out, root_ref[...]) + agg + cb_ref[...])
    gi = jnp.dot(m, wih_ref[...]) + bih_ref[...]
    gh = jnp.dot(out, whh_ref[...]) + bhh_ref[...]
    r = jax.nn.sigmoid(gi[:, 0:D] + gh[:, 0:D])
    z = jax.nn.sigmoid(gi[:, D : 2 * D] + gh[:, D : 2 * D])
    n = jnp.tanh(gi[:, 2 * D : 3 * D] + r * gh[:, 2 * D : 3 * D])
    new_ref[:, 0:D] = (1.0 - z) * n + z * out
    new_ref[:, D:DP] = jnp.zeros((NBLK, D), jnp.float32)


def _gru(out, aggp, rcp, conv_root, conv_bias, gru_w_ih, gru_w_hh, gru_b_ih, gru_b_hh):
    nb = N // NBLK
    blk = lambda i: (i, 0)
    cst = lambda i: (0, 0)
    return pl.pallas_call(
        _gru_body,
        compiler_params=PAR,
        grid=(nb,),
        in_specs=[
            pl.BlockSpec((NBLK, DP), blk),
            pl.BlockSpec((NBLK, DP), blk),
            pl.BlockSpec((NBLK, DP), lambda i, _nb=nb: (i + _nb, 0)),
            pl.BlockSpec((NBLK, DP), blk),
            pl.BlockSpec((D, D), cst),
            pl.BlockSpec((1, D), cst),
            pl.BlockSpec((D, 3 * D), cst),
            pl.BlockSpec((D, 3 * D), cst),
            pl.BlockSpec((1, 3 * D), cst),
            pl.BlockSpec((1, 3 * D), cst),
        ],
        out_specs=pl.BlockSpec((NBLK, DP), blk),
        out_shape=jax.ShapeDtypeStruct((N, DP), jnp.float32),
    )(
        out,
        aggp,
        aggp,
        rcp,
        conv_root,
        conv_bias.reshape(1, D),
        gru_w_ih,
        gru_w_hh,
        gru_b_ih.reshape(1, 3 * D),
        gru_b_hh.reshape(1, 3 * D),
    )


NCH = 4
CR = N // NCH


def _s2s_body(
    out_ref, bcol_ref, wih_ref, whh_ref, bih_ref, bhh_ref, l1w_ref,
    l1b_ref, l2w_ref, l2b_ref, y_ref, maskf, e2_s, ex_s
):
    iota_g = lax.broadcasted_iota(jnp.int32, (CR, G), 1)

    def _build(k, carry):
        rows = pl.ds(k * CR, CR)
        maskf[rows, :] = (bcol_ref[rows, :] == iota_g).astype(jnp.float32)
        return carry

    lax.fori_loop(0, NCH, _build, 0)

    q_star = jnp.zeros((G, 2 * D), jnp.float32)
    hh = jnp.zeros((G, D), jnp.float32)
    cc = jnp.zeros((G, D), jnp.float32)
    for _ in range(3):
        gates = (
            jnp.dot(q_star, wih_ref[...])
            + bih_ref[...]
            + jnp.dot(hh, whh_ref[...])
            + bhh_ref[...]
        )
        gi = jax.nn.sigmoid(gates[:, 0:D])
        gf = jax.nn.sigmoid(gates[:, D : 2 * D])
        gg = jnp.tanh(gates[:, 2 * D : 3 * D])
        go = jax.nn.sigmoid(gates[:, 3 * D : 4 * D])
        cc = gf * cc + gi * gg
        hh = go * jnp.tanh(cc)

        def _p1(k, mx):
            rows = pl.ds(k * CR, CR)
            mf = maskf[rows, :]
            qb = jnp.dot(mf, hh, precision=HI)
            e2 = jnp.sum(out_ref[rows, :] * qb, axis=1, keepdims=True)
            e2_s[rows, :] = e2
            masked = jnp.where(mf > 0.5, jnp.broadcast_to(e2, (CR, G)), -3.4e38)
            return jnp.maximum(mx, jnp.max(masked, axis=0, keepdims=True))

        mx = lax.fori_loop(0, NCH, _p1, jnp.full((1, G), -3.4e38, jnp.float32))

        def _p2(k, ssum):
            rows = pl.ds(k * CR, CR)
            mf = maskf[rows, :]
            mxb = jnp.sum(mf * mx, axis=1, keepdims=True)
            ex = jnp.exp(e2_s[rows, :] - mxb)
            ex_s[rows, :] = ex
            return ssum + jnp.sum(mf * ex, axis=0, keepdims=True)

        ssum = lax.fori_loop(0, NCH, _p2, jnp.zeros((1, G), jnp.float32))

        def _p3(k, r_read):
            rows = pl.ds(k * CR, CR)
            mf = maskf[rows, :]
            ssb = jnp.sum(mf * ssum, axis=1, keepdims=True)
            a = ex_s[rows, :] / (ssb + 1e-16)
            return r_read + lax.dot_general(
                mf, a * out_ref[rows, :], (((0,), (0,)), ((), ())), precision=HI
            )

        r_read = lax.fori_loop(0, NCH, _p3, jnp.zeros((G, D), jnp.float32))
        q_star = jnp.concatenate([hh, r_read], axis=1)

    y = jax.nn.relu(jnp.dot(q_star, l1w_ref[...]) + l1b_ref[...])
    y_ref[...] = jnp.dot(y, l2w_ref[...]) + l2b_ref[...]


def _s2s(out, batch, s2s_w_ih, s2s_w_hh, s2s_b_ih, s2s_b_hh, lin1_w, lin1_b,
         lin2_w, lin2_b):
    return pl.pallas_call(
        _s2s_body,
        out_shape=jax.ShapeDtypeStruct((G, 1), jnp.float32),
        scratch_shapes=[
            pltpu.VMEM((N, G), jnp.float32),
            pltpu.VMEM((N, 1), jnp.float32),
            pltpu.VMEM((N, 1), jnp.float32),
        ],
    )(
        out[:, 0:D],
        batch.reshape(N, 1),
        s2s_w_ih,
        s2s_w_hh,
        s2s_b_ih.reshape(1, 4 * D),
        s2s_b_hh.reshape(1, 4 * D),
        lin1_w,
        lin1_b.reshape(1, D),
        lin2_w,
        lin2_b.reshape(1, 1),
    )


def kernel(x, edge_index, edge_attr, batch, lin0_w, lin0_b, nn_w1, nn_b1,
           nn_w2, nn_b2, conv_root, conv_bias, gru_w_ih, gru_w_hh, gru_b_ih,
           gru_b_hh, s2s_w_ih, s2s_w_hh, s2s_b_ih, s2s_b_hh, lin1_w, lin1_b,
           lin2_w, lin2_b):
    src = edge_index[0]
    dst3 = edge_index[1].reshape(NW, NCHUNK, CHUNK)
    zeros_nd = jnp.zeros((N, DP), jnp.float32)

    out, weht = _pre(x, lin0_w, lin0_b, edge_attr, nn_w1, nn_b1)
    wt = _we(weht, nn_w2)
    b2t = nn_b2.reshape(D, D).T
    cntp = _sc_scatter_add(jnp.ones((E, DP), jnp.float32), dst3, zeros_nd)
    rcp = _rcp(cntp)

    for _ in range(6):
        xj = _sc_gather(out, src)
        msg = _msg(wt, xj, b2t)
        aggp = _sc_scatter_add(msg, dst3, zeros_nd)
        out = _gru(out, aggp, rcp, conv_root, conv_bias, gru_w_ih, gru_w_hh,
                   gru_b_ih, gru_b_hh)

    return _s2s(out, batch, s2s_w_ih, s2s_w_hh, s2s_b_ih, s2s_b_hh, lin1_w,
                lin1_b, lin2_w, lin2_b)
